# Initial kernel scaffold; baseline (speedup 1.0000x reference)
import jax
import jax.numpy as jnp
from jax.experimental import pallas as pl


def kernel(data, queries, truths):
    raise NotImplementedError("write your pallas kernel here")


# speedup vs baseline: 2.3969x; 2.3969x over previous
import jax
import jax.numpy as jnp
from jax.experimental import pallas as pl
from jax.experimental.pallas import tpu as pltpu

_ATOL = 1e-8
_RTOL = 1e-5


def _body(qT_ref, tT_ref, data_ref, out_ref, qn_s, tlo_s):
    i = pl.program_id(0)

    @pl.when(i == 0)
    def _init():
        q = qT_ref[...]
        t = tT_ref[...]
        qn = q / jnp.maximum(jnp.sqrt(jnp.sum(q * q, axis=0, keepdims=True)), 1e-12)
        tn = t / jnp.maximum(jnp.sqrt(jnp.sum(t * t, axis=0, keepdims=True)), 1e-12)
        thr = jnp.sum(qn * tn, axis=0, keepdims=True)
        qn_s[...] = qn
        tlo_s[...] = thr - (_ATOL + _RTOL * jnp.abs(thr))
        out_ref[...] = jnp.full(out_ref.shape, -1, jnp.int32)

    s = jnp.dot(data_ref[...], qn_s[...], preferred_element_type=jnp.float32)
    cond = s >= tlo_s[...]
    out_ref[...] += jnp.sum(cond.astype(jnp.int32), axis=0, keepdims=True)


def _row_block(n):
    for bn in range(min(n, 2048), 7, -8):
        if n % bn == 0:
            return bn
    return n


def kernel(data, queries, truths):
    n, d = data.shape
    q = queries.shape[0]
    bn = _row_block(n)
    out = pl.pallas_call(
        _body,
        grid=(n // bn,),
        in_specs=[
            pl.BlockSpec((d, q), lambda i: (0, 0)),
            pl.BlockSpec((d, q), lambda i: (0, 0)),
            pl.BlockSpec((bn, d), lambda i: (i, 0)),
        ],
        out_specs=pl.BlockSpec((1, q), lambda i: (0, 0)),
        out_shape=jax.ShapeDtypeStruct((1, q), jnp.int32),
        scratch_shapes=[
            pltpu.VMEM((d, q), jnp.float32),
            pltpu.VMEM((1, q), jnp.float32),
        ],
    )(queries.T, truths.T, data)
    return out[0]
